# baseline (speedup 1.0000x reference)
import functools

import jax
import jax.numpy as jnp
from jax import lax
from jax.experimental import pallas as pl
from jax.experimental.pallas import tpu as pltpu
from jax.experimental.pallas import tpu_sc as plsc

_B = 16384
_L = 64
_GLOBAL_BIAS = 3.5

_INFO = plsc.get_sparse_core_info()
_NC = _INFO.num_cores
_NS = _INFO.num_subcores
_LANES = _INFO.num_lanes
_NW = _NC * _NS
_BPW = _B // _NW
_GROUPS = _BPW // _LANES
_C = 128


def _mf_kernel(users_hbm, items_hbm, ue_hbm, ie_hbm, ub_hbm, ib_hbm,
               out_hbm, idx_u, idx_i, idxq_u, idxq_i, ue_v, ie_v,
               ub_v, ib_v, out_v, sem_u, sem_i, sem_ub, sem_ib):
    wid = lax.axis_index("s") * _NC + lax.axis_index("c")
    base = wid * _BPW

    pltpu.sync_copy(users_hbm.at[pl.ds(base, _BPW)], idx_u)
    pltpu.sync_copy(items_hbm.at[pl.ds(base, _BPW)], idx_i)

    copies = []
    for c in range(_BPW // _C):
        s = pl.ds(c * _C, _C)
        copies.append(pltpu.async_copy(
            ue_hbm.at[idx_u.at[s]], ue_v.at[s, :], sem_u))
        copies.append(pltpu.async_copy(
            ie_hbm.at[idx_i.at[s]], ie_v.at[s, :], sem_i))

    def quant_body(k, carry):
        s = pl.ds(k * _LANES, _LANES)
        idxq_u[s] = lax.shift_right_logical(idx_u[s], 4)
        idxq_i[s] = lax.shift_right_logical(idx_i[s], 4)
        return carry

    lax.fori_loop(0, _GROUPS, quant_body, 0)

    for c in range(_BPW // _C):
        s = pl.ds(c * _C, _C)
        copies.append(pltpu.async_copy(
            ub_hbm.at[idxq_u.at[s]], ub_v.at[s, :], sem_ub))
        copies.append(pltpu.async_copy(
            ib_hbm.at[idxq_i.at[s]], ib_v.at[s, :], sem_ib))
    for cp in copies:
        cp.wait()

    def group_body(g, carry):
        s = pl.ds(g * _LANES, _LANES)
        rows = g * _LANES + lax.iota(jnp.int32, _LANES)
        acc = jnp.zeros((_LANES,), jnp.float32)
        for j in range(_L):
            jv = jnp.full((_LANES,), j, jnp.int32)
            u = plsc.load_gather(ue_v, [rows, jv])
            v = plsc.load_gather(ie_v, [rows, jv])
            acc = acc + u * v
        um = jnp.bitwise_and(idx_u[s], 15)
        im = jnp.bitwise_and(idx_i[s], 15)
        ub = plsc.load_gather(ub_v, [rows, um])
        ib = plsc.load_gather(ib_v, [rows, im])
        out_v[s] = acc + ub + ib + _GLOBAL_BIAS
        return carry

    lax.fori_loop(0, _GROUPS, group_body, 0)

    pltpu.sync_copy(out_v, out_hbm.at[pl.ds(base, _BPW)])


@functools.partial(
    pl.kernel,
    mesh=plsc.VectorSubcoreMesh(core_axis_name="c", subcore_axis_name="s"),
    out_type=jax.ShapeDtypeStruct((_B,), jnp.float32),
    compiler_params=pltpu.CompilerParams(
        needs_layout_passes=False, use_tc_tiling_on_sc=False),
    scratch_types=[
        pltpu.VMEM((_BPW,), jnp.int32),
        pltpu.VMEM((_BPW,), jnp.int32),
        pltpu.VMEM((_BPW,), jnp.int32),
        pltpu.VMEM((_BPW,), jnp.int32),
        pltpu.VMEM((_BPW, _L), jnp.float32),
        pltpu.VMEM((_BPW, _L), jnp.float32),
        pltpu.VMEM((_BPW, _LANES), jnp.float32),
        pltpu.VMEM((_BPW, _LANES), jnp.float32),
        pltpu.VMEM((_BPW,), jnp.float32),
        pltpu.SemaphoreType.DMA,
        pltpu.SemaphoreType.DMA,
        pltpu.SemaphoreType.DMA,
        pltpu.SemaphoreType.DMA,
    ],
)
def _mf_sc(users, items, ue, ie, ub, ib, out, *scratch):
    _mf_kernel(users, items, ue, ie, ub, ib, out, *scratch)


def kernel(users, items, uEmbed, itemEmbed, uBias, itemBias):
    n_u = uBias.shape[0]
    n_i = itemBias.shape[0]
    score = _mf_sc(users.astype(jnp.int32), items.astype(jnp.int32),
                   uEmbed, itemEmbed,
                   uBias.reshape(n_u // _LANES, _LANES),
                   itemBias.reshape(n_i // _LANES, _LANES))
    return score.reshape(_B, 1)


# speedup vs baseline: 1.0010x; 1.0010x over previous
import functools

import jax
import jax.numpy as jnp
from jax import lax
from jax.experimental import pallas as pl
from jax.experimental.pallas import tpu as pltpu
from jax.experimental.pallas import tpu_sc as plsc

_B = 16384
_L = 64
_GLOBAL_BIAS = 3.5

_INFO = plsc.get_sparse_core_info()
_NC = _INFO.num_cores
_NS = _INFO.num_subcores
_LANES = _INFO.num_lanes
_NW = _NC * _NS
_BPW = _B // _NW
_GROUPS = _BPW // _LANES
_C = 128


def _mf_kernel(users_hbm, items_hbm, ue_hbm, ie_hbm, ub_hbm, ib_hbm,
               out_hbm, idx_u, idx_i, ue_v, ie_v,
               ub_v, ib_v, out_v, sem_u, sem_i, sem_ub, sem_ib):
    wid = lax.axis_index("s") * _NC + lax.axis_index("c")
    base = wid * _BPW

    pltpu.sync_copy(users_hbm.at[pl.ds(base, _BPW)], idx_u)
    pltpu.sync_copy(items_hbm.at[pl.ds(base, _BPW)], idx_i)

    copies = []
    for c in range(_BPW // _C):
        s = pl.ds(c * _C, _C)
        copies.append(pltpu.async_copy(
            ue_hbm.at[idx_u.at[s]], ue_v.at[s, :], sem_u))
        copies.append(pltpu.async_copy(
            ie_hbm.at[idx_i.at[s]], ie_v.at[s, :], sem_i))

    for c in range(_BPW // _C):
        s = pl.ds(c * _C, _C)
        copies.append(pltpu.async_copy(
            ub_hbm.at[idx_u.at[s]], ub_v.at[s], sem_ub))
        copies.append(pltpu.async_copy(
            ib_hbm.at[idx_i.at[s]], ib_v.at[s], sem_ib))
    for cp in copies:
        cp.wait()

    def group_body(g, carry):
        s = pl.ds(g * _LANES, _LANES)
        rows = g * _LANES + lax.iota(jnp.int32, _LANES)
        acc = jnp.zeros((_LANES,), jnp.float32)
        for j in range(_L):
            jv = jnp.full((_LANES,), j, jnp.int32)
            u = plsc.load_gather(ue_v, [rows, jv])
            v = plsc.load_gather(ie_v, [rows, jv])
            acc = acc + u * v
        ub = ub_v[s]
        ib = ib_v[s]
        out_v[s] = acc + ub + ib + _GLOBAL_BIAS
        return carry

    lax.fori_loop(0, _GROUPS, group_body, 0)

    pltpu.sync_copy(out_v, out_hbm.at[pl.ds(base, _BPW)])


@functools.partial(
    pl.kernel,
    mesh=plsc.VectorSubcoreMesh(core_axis_name="c", subcore_axis_name="s"),
    out_type=jax.ShapeDtypeStruct((_B,), jnp.float32),
    compiler_params=pltpu.CompilerParams(
        needs_layout_passes=False, use_tc_tiling_on_sc=False),
    scratch_types=[
        pltpu.VMEM((_BPW,), jnp.int32),
        pltpu.VMEM((_BPW,), jnp.int32),
        pltpu.VMEM((_BPW, _L), jnp.float32),
        pltpu.VMEM((_BPW, _L), jnp.float32),
        pltpu.VMEM((_BPW,), jnp.float32),
        pltpu.VMEM((_BPW,), jnp.float32),
        pltpu.VMEM((_BPW,), jnp.float32),
        pltpu.SemaphoreType.DMA,
        pltpu.SemaphoreType.DMA,
        pltpu.SemaphoreType.DMA,
        pltpu.SemaphoreType.DMA,
    ],
)
def _mf_sc(users, items, ue, ie, ub, ib, out, *scratch):
    _mf_kernel(users, items, ue, ie, ub, ib, out, *scratch)


def kernel(users, items, uEmbed, itemEmbed, uBias, itemBias):
    score = _mf_sc(users.astype(jnp.int32), items.astype(jnp.int32),
                   uEmbed, itemEmbed,
                   uBias.reshape(-1), itemBias.reshape(-1))
    return score.reshape(_B, 1)
